# baseline (speedup 1.0000x reference)
import functools

import jax
import jax.numpy as jnp
from jax import lax
from jax.experimental import pallas as pl
from jax.experimental.pallas import tpu as pltpu
from jax.experimental.pallas import tpu_sc as plsc

B = 16384
EMB = 64
L = 16
N_REL = 100

_info = plsc.get_sparse_core_info()
_NC, _NS = _info.num_cores, _info.num_subcores
NW = _NC * _NS
BPW = B // NW
NCHUNK = BPW // L
G = 64
NGRP = BPW // G
CPG = G // L

_mesh = plsc.VectorSubcoreMesh(core_axis_name="c", subcore_axis_name="s")


@functools.partial(
    pl.kernel,
    mesh=_mesh,
    compiler_params=pltpu.CompilerParams(needs_layout_passes=False),
    out_type=jax.ShapeDtypeStruct((B,), jnp.float32),
    scratch_types=[
        pltpu.VMEM((NCHUNK, L), jnp.int32),
        pltpu.VMEM((NCHUNK, L), jnp.int32),
        pltpu.VMEM((NCHUNK, L), jnp.int32),
        pltpu.VMEM((G, EMB), jnp.float32),
        pltpu.VMEM((G, EMB), jnp.float32),
        pltpu.VMEM((G, EMB), jnp.float32),
        pltpu.VMEM((G, EMB), jnp.float32),
        pltpu.VMEM((N_REL, EMB), jnp.float32),
        pltpu.VMEM((BPW,), jnp.float32),
        pltpu.SemaphoreType.DMA,
        pltpu.SemaphoreType.DMA,
        pltpu.SemaphoreType.DMA,
        pltpu.SemaphoreType.DMA,
    ],
)
def _lp_kernel(s_hbm, p_hbm, o_hbm, node_hbm, rel_hbm, out_hbm,
               pidx_v, sidx_v, oidx_v,
               sbuf0, sbuf1, obuf0, obuf1, rel_l, out_v,
               sem_s0, sem_s1, sem_o0, sem_o1):
    wid = lax.axis_index("s") * _NC + lax.axis_index("c")

    pltpu.sync_copy(s_hbm.at[wid], sidx_v)
    pltpu.sync_copy(o_hbm.at[wid], oidx_v)
    pltpu.sync_copy(p_hbm.at[wid], pidx_v)

    sbufs = (sbuf0, sbuf1)
    obufs = (obuf0, obuf1)
    ssems = (sem_s0, sem_s1)
    osems = (sem_o0, sem_o1)

    def issue(g, par):
        sb, ob = sbufs[par], obufs[par]
        sem_s, sem_o = ssems[par], osems[par]

        def dma_body(k, carry):
            c = g * CPG + k
            sv = sidx_v[c, :]
            ov = oidx_v[c, :]
            for j in range(L):
                i = k * L + j
                pltpu.async_copy(
                    node_hbm.at[pl.ds(sv[j], 1)],
                    sb.at[pl.ds(i, 1)], sem_s)
                pltpu.async_copy(
                    node_hbm.at[pl.ds(ov[j], 1)],
                    ob.at[pl.ds(i, 1)], sem_o)
            return carry

        lax.fori_loop(0, CPG, dma_body, 0)

    def wait(par):
        pltpu.make_async_copy(node_hbm.at[pl.ds(0, G)],
                              sbufs[par], ssems[par]).wait()
        pltpu.make_async_copy(node_hbm.at[pl.ds(0, G)],
                              obufs[par], osems[par]).wait()

    lane = lax.iota(jnp.int32, L)

    def compute(g, par):
        sb, ob = sbufs[par], obufs[par]
        for k in range(CPG):
            c_static_off = k
            c = g * CPG + c_static_off
            trip = k * L + lane
            pvec = pidx_v[c, :]
            acc = jnp.zeros((L,), jnp.float32)
            for d in range(EMB):
                sv = plsc.load_gather(sb, [trip, lane * 0 + d])
                ov = plsc.load_gather(ob, [trip, lane * 0 + d])
                pv = plsc.load_gather(rel_l, [pvec, lane * 0 + d])
                acc = acc + sv * pv * ov
            out_v[pl.ds(c * L, L)] = 1.0 / (1.0 + jnp.exp(-acc))

    issue(0, 0)
    pltpu.sync_copy(rel_hbm, rel_l)

    def pair_body(h, carry):
        g0 = 2 * h
        issue(g0 + 1, 1)
        wait(0)
        compute(g0, 0)

        @pl.when(h < NGRP // 2 - 1)
        def _():
            issue(g0 + 2, 0)

        wait(1)
        compute(g0 + 1, 1)
        return carry

    lax.fori_loop(0, NGRP // 2, pair_body, 0)

    pltpu.sync_copy(out_v, out_hbm.at[pl.ds(wid * BPW, BPW)])


def kernel(s_idx, p_idx, o_idx, node_emb, rel_emb):
    s3 = s_idx.reshape(NW, NCHUNK, L)
    o3 = o_idx.reshape(NW, NCHUNK, L)
    p3 = p_idx.reshape(NW, NCHUNK, L)
    return _lp_kernel(s3, p3, o3, node_emb, rel_emb)


# speedup vs baseline: 1.0068x; 1.0068x over previous
import functools

import jax
import jax.numpy as jnp
from jax import lax
from jax.experimental import pallas as pl
from jax.experimental.pallas import tpu as pltpu
from jax.experimental.pallas import tpu_sc as plsc

B = 16384
EMB = 64
L = 16
N_REL = 100
N_NODES = 1000000

_info = plsc.get_sparse_core_info()
_NC, _NS = _info.num_cores, _info.num_subcores
NW = _NC * _NS
BPW = B // NW
NCHUNK = BPW // L
NGRP = BPW // 128
CPG = 128 // L

_mesh = plsc.VectorSubcoreMesh(core_axis_name="c", subcore_axis_name="s")


@functools.partial(
    pl.kernel,
    mesh=_mesh,
    compiler_params=pltpu.CompilerParams(needs_layout_passes=False),
    out_type=jax.ShapeDtypeStruct((B,), jnp.float32),
    scratch_types=[
        pltpu.VMEM((NGRP, 128), jnp.int32),
        pltpu.VMEM((NGRP, 128), jnp.int32),
        pltpu.VMEM((NCHUNK, L), jnp.int32),
        pltpu.VMEM((16, 8, EMB), jnp.float32),
        pltpu.VMEM((16, 8, EMB), jnp.float32),
        pltpu.VMEM((16, 8, EMB), jnp.float32),
        pltpu.VMEM((16, 8, EMB), jnp.float32),
        pltpu.VMEM((N_REL, EMB), jnp.float32),
        pltpu.VMEM((BPW,), jnp.float32),
        pltpu.SemaphoreType.DMA,
        pltpu.SemaphoreType.DMA,
        pltpu.SemaphoreType.DMA,
        pltpu.SemaphoreType.DMA,
    ],
)
def _lp_kernel(s_hbm, p_hbm, o_hbm, node_hbm, rel_hbm, out_hbm,
               sidx_v, oidx_v, pidx_v,
               sbuf0, sbuf1, obuf0, obuf1, rel_l, out_v,
               sem_s0, sem_s1, sem_o0, sem_o1):
    wid = lax.axis_index("s") * _NC + lax.axis_index("c")
    node3 = node_hbm.reshape(N_NODES // 8, 8, EMB)

    pltpu.sync_copy(s_hbm.at[wid], sidx_v)
    pltpu.sync_copy(o_hbm.at[wid], oidx_v)
    pltpu.sync_copy(p_hbm.at[wid], pidx_v)

    sbufs = (sbuf0, sbuf1)
    obufs = (obuf0, obuf1)
    ssems = (sem_s0, sem_s1)
    osems = (sem_o0, sem_o1)

    def issue(g):
        sb, ob = sbufs[g % 2], obufs[g % 2]
        sem_s, sem_o = ssems[g % 2], osems[g % 2]

        def dma_body(k, carry):
            sv = sidx_v[g, pl.ds(k * L, L)]
            ov = oidx_v[g, pl.ds(k * L, L)]
            for j in range(L):
                i = k * L + j
                rs = sv[j]
                pltpu.async_copy(
                    node3.at[pl.ds(rs >> 3, 1), pl.ds(rs & 7, 1)],
                    sb.at[pl.ds(i >> 3, 1), pl.ds(i & 7, 1)], sem_s)
                ro = ov[j]
                pltpu.async_copy(
                    node3.at[pl.ds(ro >> 3, 1), pl.ds(ro & 7, 1)],
                    ob.at[pl.ds(i >> 3, 1), pl.ds(i & 7, 1)], sem_o)
            return carry

        lax.fori_loop(0, CPG, dma_body, 0)

    def wait(par):
        pltpu.make_async_copy(node3.at[pl.ds(0, 16)],
                              sbufs[par], ssems[par]).wait()
        pltpu.make_async_copy(node3.at[pl.ds(0, 16)],
                              obufs[par], osems[par]).wait()

    issue(0)
    pltpu.sync_copy(rel_hbm, rel_l)

    lane = lax.iota(jnp.int32, L)

    for g in range(NGRP):
        if g + 1 < NGRP:
            issue(g + 1)
        wait(g % 2)
        sb, ob = sbufs[g % 2], obufs[g % 2]

        def chunk_body(lc, carry, g=g, sb=sb, ob=ob):
            c = g * CPG + lc
            rows = lc * L + lane
            tq = rows >> 3
            tr = rows & 7
            pvec = pidx_v[c, :]
            acc = jnp.zeros((L,), jnp.float32)
            for d in range(EMB):
                sv = plsc.load_gather(sb, [tq, tr, lane * 0 + d])
                ov = plsc.load_gather(ob, [tq, tr, lane * 0 + d])
                pv = plsc.load_gather(rel_l, [pvec, lane * 0 + d])
                acc = acc + sv * pv * ov
            out_v[pl.ds(c * L, L)] = 1.0 / (1.0 + jnp.exp(-acc))
            return carry

        lax.fori_loop(0, CPG, chunk_body, 0)

    pltpu.sync_copy(out_v, out_hbm.at[pl.ds(wid * BPW, BPW)])


def kernel(s_idx, p_idx, o_idx, node_emb, rel_emb):
    s3 = s_idx.reshape(NW, NGRP, 128)
    o3 = o_idx.reshape(NW, NGRP, 128)
    p3 = p_idx.reshape(NW, NCHUNK, L)
    return _lp_kernel(s3, p3, o3, node_emb, rel_emb)
